# baseline (speedup 1.0000x reference)
import functools

import jax
import jax.numpy as jnp
from jax import lax
from jax.experimental import pallas as pl
from jax.experimental.pallas import tpu as pltpu
from jax.experimental.pallas import tpu_sc as plsc

D4 = 512
N_TOK = 16384
N_TAB = 4
NC, NS, L = 2, 16, 16
NW = NC * NS
TOK_PER_W = N_TOK // NW
C_TOK = 16
ROWS_PER_STEP = C_TOK * N_TAB
N_STEP = TOK_PER_W // C_TOK


def _sc_gather(x_i32, table):
    mesh = plsc.VectorSubcoreMesh(core_axis_name="c", subcore_axis_name="s")

    @functools.partial(
        pl.kernel,
        mesh=mesh,
        compiler_params=pltpu.CompilerParams(needs_layout_passes=False),
        out_type=jax.ShapeDtypeStruct((N_TOK * N_TAB, D4), jnp.float32),
        scratch_types=[
            pltpu.VMEM((TOK_PER_W,), jnp.int32),
            pltpu.VMEM((TOK_PER_W * N_TAB,), jnp.int32),
            pltpu.VMEM((2, ROWS_PER_STEP, D4), jnp.float32),
            pltpu.SemaphoreType.DMA,
            pltpu.SemaphoreType.DMA,
            pltpu.SemaphoreType.DMA,
            pltpu.SemaphoreType.DMA,
        ],
    )
    def k(x_hbm, tab_hbm, out_hbm, x_v, idx_v, rows_v, gs0, gs1, ss0, ss1):
        wid = lax.axis_index("s") * NC + lax.axis_index("c")
        tok_base = wid * TOK_PER_W

        pltpu.sync_copy(x_hbm.at[pl.ds(tok_base, TOK_PER_W)], x_v)

        lane = lax.iota(jnp.int32, 16)
        pos0 = lane * N_TAB

        def build(g, carry):
            v = x_v[pl.ds(g * 16, 16)]
            base = g * (16 * N_TAB)
            for kk in range(N_TAB):
                b = lax.shift_right_logical(v, 8 * kk) & 255
                plsc.store_scatter(idx_v, [base + pos0 + kk], b + kk * 256)
            return carry

        lax.fori_loop(0, TOK_PER_W // 16, build, 0)

        gsems = (gs0, gs1)
        ssems = (ss0, ss1)

        def start_gather(step, buf):
            idx_slice = idx_v.at[pl.ds(step * ROWS_PER_STEP, ROWS_PER_STEP)]
            return pltpu.async_copy(
                tab_hbm.at[idx_slice], rows_v.at[buf], gsems[buf]
            )

        def start_scatter(step, buf):
            row0 = tok_base * N_TAB + step * ROWS_PER_STEP
            return pltpu.async_copy(
                rows_v.at[buf], out_hbm.at[pl.ds(row0, ROWS_PER_STEP)], ssems[buf]
            )

        gh = [None, None]
        sh = [None, None]
        gh[0] = start_gather(0, 0)
        gh[1] = start_gather(1, 1)
        for step in range(N_STEP):
            buf = step & 1
            gh[buf].wait()
            sh[buf] = start_scatter(step, buf)
            if step + 2 < N_STEP:
                sh[buf].wait()
                gh[buf] = start_gather(step + 2, buf)
        sh[0].wait()
        sh[1].wait()

    return k(x_i32, table)


@jax.jit
def kernel(x, W1, W2, W3, W4):
    table = jnp.concatenate([W1, W2, W3, W4], axis=0)
    x_i32 = lax.bitcast_convert_type(x.reshape(-1), jnp.int32)
    out = _sc_gather(x_i32, table)
    return out.reshape(x.shape[0], x.shape[1], N_TAB * D4)


# speedup vs baseline: 1.0356x; 1.0356x over previous
import functools

import jax
import jax.numpy as jnp
from jax import lax
from jax.experimental import pallas as pl
from jax.experimental.pallas import tpu as pltpu
from jax.experimental.pallas import tpu_sc as plsc

D4 = 512
N_TOK = 16384
N_TAB = 4
NC, NS, L = 2, 16, 16
NW = NC * NS
TOK_PER_W = N_TOK // NW
C_TOK = 8
ROWS_PER_STEP = C_TOK * N_TAB
N_STEP = TOK_PER_W // C_TOK
NBUF = 4


def _sc_gather(x_i32, table):
    mesh = plsc.VectorSubcoreMesh(core_axis_name="c", subcore_axis_name="s")

    @functools.partial(
        pl.kernel,
        mesh=mesh,
        compiler_params=pltpu.CompilerParams(needs_layout_passes=False),
        out_type=jax.ShapeDtypeStruct((N_TOK * N_TAB, D4), jnp.float32),
        scratch_types=[
            pltpu.VMEM((TOK_PER_W,), jnp.int32),
            pltpu.VMEM((TOK_PER_W * N_TAB,), jnp.int32),
            pltpu.VMEM((NBUF, ROWS_PER_STEP, D4), jnp.float32),
            [pltpu.SemaphoreType.DMA] * NBUF,
            [pltpu.SemaphoreType.DMA] * NBUF,
        ],
    )
    def k(x_hbm, tab_hbm, out_hbm, x_v, idx_v, rows_v, gsems, ssems):
        wid = lax.axis_index("s") * NC + lax.axis_index("c")
        tok_base = wid * TOK_PER_W

        pltpu.sync_copy(x_hbm.at[pl.ds(tok_base, TOK_PER_W)], x_v)

        lane = lax.iota(jnp.int32, 16)
        pos0 = lane * N_TAB

        def build(g, carry):
            v = x_v[pl.ds(g * 16, 16)]
            base = g * (16 * N_TAB)
            for kk in range(N_TAB):
                b = lax.shift_right_logical(v, 8 * kk) & 255
                plsc.store_scatter(idx_v, [base + pos0 + kk], b + kk * 256)
            return carry

        lax.fori_loop(0, TOK_PER_W // 16, build, 0)

        def start_gather(step, buf):
            idx_slice = idx_v.at[pl.ds(step * ROWS_PER_STEP, ROWS_PER_STEP)]
            return pltpu.async_copy(
                tab_hbm.at[idx_slice], rows_v.at[buf], gsems[buf]
            )

        def start_scatter(step, buf):
            row0 = tok_base * N_TAB + step * ROWS_PER_STEP
            return pltpu.async_copy(
                rows_v.at[buf], out_hbm.at[pl.ds(row0, ROWS_PER_STEP)], ssems[buf]
            )

        gh = [start_gather(b, b) for b in range(NBUF)]
        sh = [None] * NBUF
        for step in range(N_STEP):
            buf = step % NBUF
            gh[buf].wait()
            sh[buf] = start_scatter(step, buf)
            if step + NBUF < N_STEP:
                sh[buf].wait()
                gh[buf] = start_gather(step + NBUF, buf)
        for buf in range(NBUF):
            sh[(N_STEP - NBUF + buf) % NBUF].wait()

    return k(x_i32, table)


@jax.jit
def kernel(x, W1, W2, W3, W4):
    table = jnp.concatenate([W1, W2, W3, W4], axis=0)
    x_i32 = lax.bitcast_convert_type(x.reshape(-1), jnp.int32)
    out = _sc_gather(x_i32, table)
    return out.reshape(x.shape[0], x.shape[1], N_TAB * D4)
